# baseline (speedup 1.0000x reference)
import functools

import jax
import jax.numpy as jnp
from jax import lax
from jax.experimental import pallas as pl
from jax.experimental.pallas import tpu as pltpu
from jax.experimental.pallas import tpu_sc as plsc

_B = 4096
_N = 20
_D = 64
_V = 1000000
_NW = 32
_BPW = _B // _NW
_TCT = (_V + 127) // 128
_CPW = 245
_CC = 5
_NCH = _CPW // _CC
_CAP = 4096
_CCAP = 128
_DEAD = _B * _N
_SCORES = _DEAD + _CCAP

_PARAMS = pltpu.CompilerParams(
    use_tc_tiling_on_sc=True,
    needs_layout_passes=False,
)
_MESH = plsc.VectorSubcoreMesh(core_axis_name="c", subcore_axis_name="s")


def _wid():
    return lax.axis_index("s") * 2 + lax.axis_index("c")


def _bc(x):
    return lax.broadcast_in_dim(x, (16,), ())


def _make_ctx_kernel():
    @functools.partial(
        pl.kernel,
        out_type=jax.ShapeDtypeStruct((_B // 2, 128), jnp.float32),
        mesh=_MESH,
        scratch_types=[
            pltpu.VMEM((_D, _BPW), jnp.float32),
            pltpu.VMEM((_BPW // 2, 128), jnp.float32),
        ],
        compiler_params=_PARAMS,
    )
    def ctx_kernel(ctx_hbm, out_hbm, cblk, oblk):
        wid = _wid()
        w0 = wid * _BPW
        lanes = lax.iota(jnp.int32, 16)
        pltpu.sync_copy(ctx_hbm.at[:, pl.ds(w0, _BPW)], cblk)
        rowv = [lax.shift_right_logical(lanes + k * 16, 1) for k in range(8)]
        colb = [lax.shift_left((lanes + k * 16) & 1, 6) for k in range(8)]
        for d in range(_D):
            for k in range(8):
                v = cblk[d, pl.ds(k * 16, 16)]
                plsc.store_scatter(oblk, [rowv[k], colb[k] + d], v)
        pltpu.sync_copy(oblk, out_hbm.at[pl.ds(wid * (_BPW // 2), _BPW // 2)])

    return ctx_kernel


def _make_main_kernel():
    @functools.partial(
        pl.kernel,
        out_type=jax.ShapeDtypeStruct((_SCORES,), jnp.float32),
        mesh=_MESH,
        scratch_types=[
            pltpu.VMEM((_N, 128), jnp.int32),
            pltpu.VMEM((_CAP // 16, 16), jnp.int32),
            pltpu.VMEM((_CAP // 16, 16), jnp.int32),
            pltpu.VMEM((_CC, _D, 128), jnp.float32),
            pltpu.VMEM((1, 128), jnp.int32),
            pltpu.VMEM((1, 128), jnp.int32),
            pltpu.VMEM((1, 128), jnp.int32),
            pltpu.VMEM((_CCAP, 128), jnp.float32),
            pltpu.VMEM((1, 128), jnp.float32),
            pltpu.SemaphoreType.DMA,
            pltpu.SemaphoreType.DMA,
            pltpu.SemaphoreType.DMA,
        ],
        compiler_params=_PARAMS,
    )
    def main_kernel(ids_hbm, table_hbm, ctx2_hbm, scores_hbm,
                    idsb, elist, plist, tab3, cel, cpl, cbrow,
                    ctxrows, caccs, semg, semc, sems):
        wid = _wid()
        lanes = lax.iota(jnp.int32, 16)
        zi = jnp.zeros((16,), jnp.int32)
        zf = jnp.zeros((16,), jnp.float32)
        c0 = wid * _CPW
        c1 = jnp.minimum(c0 + _CPW, _TCT)
        c0v = _bc(c0)
        c1v = _bc(c1)

        def blk_body(blk, countv):
            pltpu.sync_copy(ids_hbm.at[:, pl.ds(blk * 128, 128)], idsb)

            def n_body(n, countv):
                for k in range(8):
                    ev = idsb[n, pl.ds(k * 16, 16)]
                    cv = lax.shift_right_logical(ev, 7)
                    msk = (cv >= c0v) & (cv < c1v)
                    cs = plsc.cumsum(msk.astype(jnp.int32))
                    pos = countv + cs - 1
                    pr = lax.shift_right_logical(pos, 4)
                    pc = pos & 15
                    plsc.store_scatter(elist, [pr, pc], ev, mask=msk)
                    pv = lanes + (n * _B + blk * 128 + k * 16)
                    plsc.store_scatter(plist, [pr, pc], pv, mask=msk)
                    countv = countv + plsc.all_reduce_population_count(msk)
                return countv

            return lax.fori_loop(0, _N, n_body, countv)

        countv = lax.fori_loop(0, 32, blk_body, zi)
        nvec = lax.shift_right_logical(countv[0] + 15, 4)

        def ch_body(ch, carry):
            c_lo = c0 + ch * _CC
            vcols = jnp.clip(c1 - c_lo, 0, _CC)

            @pl.when(vcols > 0)
            def _chunk():
                def t_issue(t, carry):
                    @pl.when(t < vcols)
                    def _():
                        col0 = (c_lo + t) * 128

                        @pl.when(col0 + 128 <= _V)
                        def _():
                            for r in range(8):
                                pltpu.async_copy(
                                    table_hbm.at[pl.ds(r * 8, 8),
                                                 pl.ds(col0, 128)],
                                    tab3.at[t, pl.ds(r * 8, 8)],
                                    semg,
                                )

                        @pl.when(col0 + 128 > _V)
                        def _():
                            for r in range(8):
                                pltpu.async_copy(
                                    table_hbm.at[pl.ds(r * 8, 8),
                                                 pl.ds(col0, 64)],
                                    tab3.at[t, pl.ds(r * 8, 8),
                                            pl.ds(0, 64)],
                                    semg,
                                )
                    return carry

                lax.fori_loop(0, _CC, t_issue, 0)

                eb = c_lo * 128
                ehi = (c_lo + vcols) * 128
                ebv = _bc(eb)
                ehiv = _bc(ehi)
                for k in range(8):
                    sl = pl.ds(k * 16, 16)
                    cel[0, sl] = ebv
                    cpl[0, sl] = lanes + (_DEAD + k * 16)
                    cbrow[0, sl] = zi
                    caccs[0, sl] = zf

                def rv_body(iv, ccnt):
                    ev = elist[iv, :]
                    pvv = plist[iv, :]
                    lanemsk = (lanes + _bc(iv * 16)) < countv
                    msk = (ev >= ebv) & (ev < ehiv) & lanemsk
                    cs = plsc.cumsum(msk.astype(jnp.int32))
                    pos = ccnt + cs - 1
                    pr = lax.shift_right_logical(pos, 7)
                    pc = pos & 127
                    plsc.store_scatter(cel, [pr, pc], ev, mask=msk)
                    plsc.store_scatter(cpl, [pr, pc], pvv, mask=msk)
                    brow = lax.shift_right_logical(pvv & (_B - 1), 1)
                    plsc.store_scatter(cbrow, [pr, pc], brow, mask=msk)
                    return ccnt + plsc.all_reduce_population_count(msk)

                ccntv = lax.fori_loop(0, nvec, rv_body, zi)

                ctx_cps = [
                    pltpu.async_copy(
                        ctx2_hbm.at[cbrow.at[0]],
                        ctxrows,
                        semc,
                    )
                ]

                def t_drain(t, carry):
                    @pl.when(t < vcols)
                    def _():
                        col0 = (c_lo + t) * 128

                        @pl.when(col0 + 128 <= _V)
                        def _():
                            for r in range(8):
                                pltpu.make_async_copy(
                                    table_hbm.at[pl.ds(r * 8, 8),
                                                 pl.ds(col0, 128)],
                                    tab3.at[t, pl.ds(r * 8, 8)],
                                    semg,
                                ).wait()

                        @pl.when(col0 + 128 > _V)
                        def _():
                            for r in range(8):
                                pltpu.make_async_copy(
                                    table_hbm.at[pl.ds(r * 8, 8),
                                                 pl.ds(col0, 64)],
                                    tab3.at[t, pl.ds(r * 8, 8),
                                            pl.ds(0, 64)],
                                    semg,
                                ).wait()
                    return carry

                lax.fori_loop(0, _CC, t_drain, 0)
                for cp in ctx_cps:
                    cp.wait()

                ncvec = lax.shift_right_logical(ccntv[0] + 15, 4)
                c_lov = _bc(c_lo)

                def ev_body(iv2, carry):
                    slotv = lanes + _bc(iv2 * 16)
                    sr = lax.shift_right_logical(slotv, 7)
                    sc = slotv & 127
                    evv = plsc.load_gather(cel, [sr, sc])
                    pvv = plsc.load_gather(cpl, [sr, sc])
                    tcv = lax.shift_right_logical(evv, 7) - c_lov
                    elow = evv & 127
                    halfc = lax.shift_left(pvv & 1, 6)

                    def d_body(d, acc):
                        dv = _bc(d)
                        tv = plsc.load_gather(tab3, [tcv, dv, elow])
                        cxv = plsc.load_gather(ctxrows, [slotv, halfc + dv])
                        return acc + tv * cxv

                    acc = lax.fori_loop(0, _D, d_body, zf)
                    plsc.store_scatter(caccs, [sr, sc], acc)
                    return carry

                lax.fori_loop(0, ncvec, ev_body, 0)

                pltpu.async_copy(
                    caccs.at[0],
                    scores_hbm.at[cpl.at[0]],
                    sems,
                ).wait()

            return carry

        lax.fori_loop(0, _NCH, ch_body, 0)

    return main_kernel


def _make_softmax_kernel():
    @functools.partial(
        pl.kernel,
        out_type=jax.ShapeDtypeStruct((_N, _B), jnp.float32),
        mesh=_MESH,
        scratch_types=[
            pltpu.VMEM((_N, _BPW), jnp.float32),
            pltpu.VMEM((_N, _BPW), jnp.float32),
        ],
        compiler_params=_PARAMS,
    )
    def sm_kernel(scores_hbm, out_hbm, smv, out2):
        wid = _wid()
        w0 = wid * _BPW
        for n in range(_N):
            pltpu.sync_copy(
                scores_hbm.at[pl.ds(n * _B + w0, _BPW)], smv.at[n]
            )
        for k in range(_BPW // 16):
            sl = pl.ds(k * 16, 16)
            m = smv[0, sl]
            for n in range(1, _N):
                m = jnp.maximum(m, smv[n, sl])
            tot = jnp.zeros((16,), jnp.float32)
            for n in range(_N):
                e = jnp.exp(smv[n, sl] - m)
                tot = tot + e
                out2[n, sl] = e
            for n in range(_N):
                out2[n, sl] = out2[n, sl] / tot
        pltpu.sync_copy(out2, out_hbm.at[:, pl.ds(w0, _BPW)])

    return sm_kernel


_CTX_KERNEL = _make_ctx_kernel()
_MAIN_KERNEL = _make_main_kernel()
_SM_KERNEL = _make_softmax_kernel()


def kernel(context_encoded, entity_ids, entity_embeddings):
    ctx2 = _CTX_KERNEL(context_encoded.T)
    scores = _MAIN_KERNEL(entity_ids.T, entity_embeddings.T, ctx2)
    out_t = _SM_KERNEL(scores)
    return out_t.T


# speedup vs baseline: 26.4655x; 26.4655x over previous
import functools

import jax
import jax.numpy as jnp
from jax import lax
from jax.experimental import pallas as pl
from jax.experimental.pallas import tpu as pltpu
from jax.experimental.pallas import tpu_sc as plsc

_B = 4096
_N = 20
_D = 64
_NC = 2
_NS = 16
_NW = _NC * _NS
_BPW = _B // _NW
_CHUNK = 32
_NCHUNK = _BPW // _CHUNK
_IDX = _CHUNK * _N
_G = _IDX // 128
_NEG = -1e30


def _make_sc_kernel():
    mesh = plsc.VectorSubcoreMesh(core_axis_name="c", subcore_axis_name="s")

    @functools.partial(
        pl.kernel,
        out_type=jax.ShapeDtypeStruct((_B, 32), jnp.float32),
        mesh=mesh,
        scratch_types=[
            pltpu.VMEM((_IDX,), jnp.int32),
            pltpu.VMEM((_IDX, _D), jnp.float32),
            pltpu.VMEM((_BPW, _D), jnp.float32),
            pltpu.VMEM((_CHUNK, 32), jnp.float32),
            pltpu.VMEM((32, 16), jnp.float32),
            pltpu.SemaphoreType.DMA,
        ],
        compiler_params=pltpu.CompilerParams(
            use_tc_tiling_on_sc=False,
            needs_layout_passes=False,
        ),
    )
    def sc_kernel(ctx_hbm, ids_hbm, table_hbm, out_hbm,
                  idx_v, rows_v, ctx_v, out_v, sc_v, sem):
        wid = lax.axis_index("s") * _NC + lax.axis_index("c")
        neg = jnp.full((16,), _NEG, jnp.float32)
        for n in range(_N, 32):
            sc_v[n] = neg
        pltpu.sync_copy(ctx_hbm.at[pl.ds(wid * _BPW, _BPW)], ctx_v)

        for g in range(_NCHUNK):
            b_base = wid * _BPW + g * _CHUNK
            i_base = (wid * _BPW + g * _CHUNK) * _N
            pltpu.sync_copy(ids_hbm.at[pl.ds(i_base, _IDX)], idx_v)
            copies = [
                pltpu.async_copy(
                    table_hbm.at[idx_v.at[pl.ds(j * 128, 128)]],
                    rows_v.at[pl.ds(j * 128, 128)],
                    sem,
                )
                for j in range(_G)
            ]
            for cpy in copies:
                cpy.wait()

            def body(b, carry, g=g):
                bl = g * _CHUNK + b
                c = [ctx_v[bl, pl.ds(16 * k, 16)] for k in range(4)]
                for n in range(_N):
                    r = b * _N + n
                    acc = rows_v[r, pl.ds(0, 16)] * c[0]
                    for k in range(1, 4):
                        acc = acc + rows_v[r, pl.ds(16 * k, 16)] * c[k]
                    sc_v[n] = lax.broadcast_in_dim(jnp.sum(acc), (16,), ())
                row_ids = lax.iota(jnp.int32, 16)
                col_ids = jnp.zeros((16,), jnp.int32)
                v0 = plsc.load_gather(sc_v, [row_ids, col_ids])
                v1 = plsc.load_gather(sc_v, [row_ids + 16, col_ids])
                m = jnp.maximum(jnp.max(v0), jnp.max(v1))
                e0 = jnp.exp(v0 - m)
                e1 = jnp.exp(v1 - m)
                tot = lax.broadcast_in_dim(jnp.sum(e0) + jnp.sum(e1),
                                           (16,), ())
                out_v[b, pl.ds(0, 16)] = e0 / tot
                out_v[b, pl.ds(16, 16)] = e1 / tot
                return carry

            lax.fori_loop(0, _CHUNK, body, 0)
            pltpu.sync_copy(out_v, out_hbm.at[pl.ds(b_base, _CHUNK)])

    return sc_kernel


_SC_KERNEL = _make_sc_kernel()


def kernel(context_encoded, entity_ids, entity_embeddings):
    ids_flat = entity_ids.reshape(_B * _N)
    out = _SC_KERNEL(context_encoded, ids_flat, entity_embeddings)
    return out[:, :_N]
